# baseline (speedup 1.0000x reference)
import jax
import jax.numpy as jnp
from jax import lax
from jax.experimental import pallas as pl
from jax.experimental.pallas import tpu as pltpu
from jax.experimental.pallas import tpu_sc as plsc

N = 10000
E = 320000
DIN = 128
DH = 128
DOUT = 64

NC = 2
NS = 16
EPT = E // (NC * NS)
RPT = 624
REM = N - NS * RPT
DEGW = 8

_mesh = plsc.VectorSubcoreMesh(core_axis_name="c", subcore_axis_name="s")


def _make_sc_agg(d, with_deg, eb, nbuf, ch):
    nb = EPT // eb
    nch = nb // ch
    out_type = [jax.ShapeDtypeStruct((NC * N, d), jnp.float32)]
    scratch = [
        pltpu.VMEM_SHARED((N, d), jnp.float32),
        pltpu.VMEM((2, ch, eb), jnp.int32),
        pltpu.VMEM((nb, eb), jnp.int32),
        pltpu.VMEM((nbuf, eb, d), jnp.float32),
        pltpu.SemaphoreType.DMA,
        pltpu.SemaphoreType.DMA,
    ]
    scratch += [pltpu.SemaphoreType.DMA] * (2 * nbuf)
    if with_deg:
        out_type.append(jax.ShapeDtypeStruct((NC * N, DEGW), jnp.float32))
        scratch += [
            pltpu.VMEM_SHARED((N, DEGW), jnp.float32),
            pltpu.VMEM((eb, DEGW), jnp.float32),
            pltpu.SemaphoreType.DMA,
        ]

    def sc_agg(table_hbm, edge_hbm, zrow_hbm, zdeg_hbm, ones_hbm, *rest):
        if with_deg:
            (msg_hbm, deg_hbm, acc, src_v, dst_v, rows_v, cs_s, cs_d,
             *sems) = rest
            sems, (dacc, ones_v, osem) = sems[:2 * nbuf], sems[2 * nbuf:]
        else:
            (msg_hbm, acc, src_v, dst_v, rows_v, cs_s, cs_d, *sems) = rest
        gsem = sems[:nbuf]
        ssem = sems[nbuf:2 * nbuf]
        c = lax.axis_index("c")
        s = lax.axis_index("s")
        tid = c * NS + s

        row0 = s * RPT
        pltpu.sync_copy(zrow_hbm.at[pl.ds(0, RPT)], acc.at[pl.ds(row0, RPT)])
        if with_deg:
            pltpu.sync_copy(zdeg_hbm.at[pl.ds(0, RPT)],
                            dacc.at[pl.ds(row0, RPT)])
            pltpu.sync_copy(ones_hbm, ones_v)

        @pl.when(s == NS - 1)
        def _zero_rem():
            pltpu.sync_copy(zrow_hbm.at[pl.ds(0, REM)],
                            acc.at[pl.ds(NS * RPT, REM)])
            if with_deg:
                pltpu.sync_copy(zdeg_hbm.at[pl.ds(0, REM)],
                                dacc.at[pl.ds(NS * RPT, REM)])

        plsc.subcore_barrier()

        pend_dst = pltpu.async_copy(edge_hbm.at[1, tid], dst_v, cs_d)
        pltpu.sync_copy(edge_hbm.at[0, tid, pl.ds(0, ch)], src_v.at[0])

        pend_scatter = [None] * nbuf
        pend_gather = [None] * nbuf
        pend_chunk = None
        chunk_ready = 0
        pend_ones = None

        def fire_gather(gn):
            nonlocal pend_chunk, chunk_ready
            kn = gn // ch
            if kn > chunk_ready:
                pend_chunk.wait()
                pend_chunk = None
                chunk_ready = kn
            bn = gn % nbuf
            if pend_scatter[bn] is not None:
                pend_scatter[bn].wait()
                pend_scatter[bn] = None
            pend_gather[bn] = pltpu.async_copy(
                table_hbm.at[src_v.at[kn % 2].at[gn % ch]], rows_v.at[bn],
                gsem[bn])

        for g in range(nb):
            k, j = divmod(g, ch)
            b = g % nbuf
            if j == 0 and k + 1 < nch:
                pend_chunk = pltpu.async_copy(
                    edge_hbm.at[0, tid, pl.ds((k + 1) * ch, ch)],
                    src_v.at[(k + 1) % 2], cs_s)
            if pend_gather[b] is None:
                fire_gather(g)
            if g == 0:
                pend_dst.wait()
            pend_gather[b].wait()
            pend_gather[b] = None
            for a in range(1, nbuf):
                gn = g + a
                if (gn < nb and gn // ch <= k + 1
                        and pend_gather[gn % nbuf] is None):
                    fire_gather(gn)
            pend_scatter[b] = pltpu.async_copy(
                rows_v.at[b], acc.at[dst_v.at[g]], ssem[b], add=True)
            if with_deg:
                if pend_ones is not None:
                    pend_ones.wait()
                pend_ones = pltpu.async_copy(
                    ones_v, dacc.at[dst_v.at[g]], osem, add=True)
        for p in pend_scatter:
            if p is not None:
                p.wait()
        if pend_ones is not None:
            pend_ones.wait()
        plsc.subcore_barrier()

        out0 = c * N + row0
        pltpu.sync_copy(acc.at[pl.ds(row0, RPT)], msg_hbm.at[pl.ds(out0, RPT)])
        if with_deg:
            pltpu.sync_copy(dacc.at[pl.ds(row0, RPT)],
                            deg_hbm.at[pl.ds(out0, RPT)])

        @pl.when(s == NS - 1)
        def _drain_rem():
            pltpu.sync_copy(acc.at[pl.ds(NS * RPT, REM)],
                            msg_hbm.at[pl.ds(c * N + NS * RPT, REM)])
            if with_deg:
                pltpu.sync_copy(dacc.at[pl.ds(NS * RPT, REM)],
                                deg_hbm.at[pl.ds(c * N + NS * RPT, REM)])

    return pl.kernel(
        sc_agg, mesh=_mesh, out_type=out_type, scratch_types=scratch,
        compiler_params=pltpu.CompilerParams(use_tc_tiling_on_sc=False))


EB, CH = 80, 5
NBUF1 = 3
NBUF2 = 8
_sc_agg128 = _make_sc_agg(DIN, True, EB, NBUF1, CH)
_sc_agg64 = _make_sc_agg(DOUT, False, EB, NBUF2, CH)

BN = 2000
BNP = BN // 2
PDB = BN * DEGW // 128
PDROWS = N * DEGW // 128
PYB = BN * DOUT // 128


def _deg_pair_cols(deg_ref, i):
    d = (deg_ref[pl.ds(i * PDB, PDB), :]
         + deg_ref[pl.ds(PDROWS + i * PDB, PDB), :])
    r_ = lax.broadcasted_iota(jnp.int32, (BNP, PDB), 0)
    c_ = lax.broadcasted_iota(jnp.int32, (BNP, PDB), 1)
    sel = jnp.where(c_ == r_ // 8, 1.0, 0.0)
    t = lax.dot_general(sel, d, (((1,), (0,)), ((), ())),
                        preferred_element_type=jnp.float32)
    rr = lax.broadcasted_iota(jnp.int32, (BNP, 128), 0)
    ll = lax.broadcasted_iota(jnp.int32, (BNP, 128), 1)
    de = jnp.sum(jnp.where(ll == (rr % 8) * 16, t, 0.0), axis=1,
                 keepdims=True)
    do = jnp.sum(jnp.where(ll == (rr % 8) * 16 + 8, t, 0.0), axis=1,
                 keepdims=True)
    return de, do


def _deg_col(deg_ref, i):
    d = (deg_ref[pl.ds(i * PDB, PDB), :]
         + deg_ref[pl.ds(PDROWS + i * PDB, PDB), :])
    r_ = lax.broadcasted_iota(jnp.int32, (BN, PDB), 0)
    c_ = lax.broadcasted_iota(jnp.int32, (BN, PDB), 1)
    sel = jnp.where(c_ == r_ // 16, 1.0, 0.0)
    t = lax.dot_general(sel, d, (((1,), (0,)), ((), ())),
                        preferred_element_type=jnp.float32)
    rr = lax.broadcasted_iota(jnp.int32, (BN, 128), 0)
    ll = lax.broadcasted_iota(jnp.int32, (BN, 128), 1)
    return jnp.sum(jnp.where(ll == (rr % 16) * 8, t, 0.0), axis=1,
                   keepdims=True)


def _tc_dense_body(msg_a, msg_b, deg_ref, x_ref, wl_ref, bl_ref,
                   wr_ref, wg_ref, y_ref):
    m = msg_a[...] + msg_b[...]
    deg = _deg_col(deg_ref, pl.program_id(0))
    mean = m / jnp.maximum(deg, 1.0)
    dn = (((1,), (1,)), ((), ()))
    h = lax.dot_general(mean, wl_ref[...], dn,
                        preferred_element_type=jnp.float32)
    h = h + bl_ref[...]
    h = h + lax.dot_general(x_ref[...], wr_ref[...], dn,
                            preferred_element_type=jnp.float32)
    h = jnp.maximum(h, 0.0)
    xt = lax.dot_general(h, wg_ref[...], dn,
                         preferred_element_type=jnp.float32)
    y_ref[...] = lax.rsqrt(deg + 1.0) * xt


_tc_dense = pl.pallas_call(
    _tc_dense_body,
    grid=(N // BN,),
    in_specs=[
        pl.BlockSpec((BN, DIN), lambda i: (i, 0)),
        pl.BlockSpec((BN, DIN), lambda i: (N // BN + i, 0)),
        pl.BlockSpec((NC * PDROWS, 128), lambda i: (0, 0)),
        pl.BlockSpec((BN, DIN), lambda i: (i, 0)),
        pl.BlockSpec((DH, DIN), lambda i: (0, 0)),
        pl.BlockSpec((1, DH), lambda i: (0, 0)),
        pl.BlockSpec((DH, DIN), lambda i: (0, 0)),
        pl.BlockSpec((DOUT, DH), lambda i: (0, 0)),
    ],
    out_specs=pl.BlockSpec((BN, DOUT), lambda i: (i, 0)),
    out_shape=jax.ShapeDtypeStruct((N, DOUT), jnp.float32),
)


def _tc_final_body(agg_a, agg_b, deg_ref, y_ref, bg_ref,
                   out_ref, soft_ref):
    i = pl.program_id(0)
    a2 = agg_a[...] + agg_b[...] + y_ref[...]
    de, do = _deg_pair_cols(deg_ref, i)
    outs, softs = [], []
    for half, dg in ((0, de), (1, do)):
        a = a2[:, half * DOUT:(half + 1) * DOUT]
        out = lax.rsqrt(dg + 1.0) * a + bg_ref[...]
        m = jnp.max(out, axis=1, keepdims=True)
        e = jnp.exp(out - m)
        outs.append(out)
        softs.append(e / jnp.sum(e, axis=1, keepdims=True))
    out_ref[...] = jnp.concatenate(outs, axis=1)
    soft_ref[...] = jnp.concatenate(softs, axis=1)


_tc_final = pl.pallas_call(
    _tc_final_body,
    grid=(N // BN,),
    in_specs=[
        pl.BlockSpec((PYB, 128), lambda i: (i, 0)),
        pl.BlockSpec((PYB, 128), lambda i: (N // BN + i, 0)),
        pl.BlockSpec((NC * PDROWS, 128), lambda i: (0, 0)),
        pl.BlockSpec((PYB, 128), lambda i: (i, 0)),
        pl.BlockSpec((1, DOUT), lambda i: (0, 0)),
    ],
    out_specs=[
        pl.BlockSpec((PYB, 128), lambda i: (i, 0)),
        pl.BlockSpec((PYB, 128), lambda i: (i, 0)),
    ],
    out_shape=[
        jax.ShapeDtypeStruct((N * DOUT // 128, 128), jnp.float32),
        jax.ShapeDtypeStruct((N * DOUT // 128, 128), jnp.float32),
    ],
)


def kernel(x, edge_index, W_sage_l, b_sage_l, W_sage_r, W_gcn, b_gcn):
    e4 = edge_index.reshape(2, NC * NS, EPT // EB, EB)
    zrow = jnp.zeros((RPT, DIN), jnp.float32)
    zdeg = jnp.zeros((RPT, DEGW), jnp.float32)
    ones = jnp.ones((EB, DEGW), jnp.float32)
    msg2, deg2 = _sc_agg128(x, e4, zrow, zdeg, ones)
    degp = deg2.reshape(NC * PDROWS, 128)
    y = _tc_dense(msg2, msg2, degp, x, W_sage_l,
                  b_sage_l.reshape(1, DH), W_sage_r, W_gcn)
    zrow64 = jnp.zeros((RPT, DOUT), jnp.float32)
    agg2 = _sc_agg64(y, e4, zrow64, zdeg, ones)[0]
    aggp = agg2.reshape(NC * N * DOUT // 128, 128)
    outp, softp = _tc_final(aggp, aggp, degp, y.reshape(N // 2, 128),
                            b_gcn.reshape(1, DOUT))
    return outp.reshape(N, DOUT), softp.reshape(N, DOUT)


# speedup vs baseline: 1.0073x; 1.0073x over previous
import jax
import jax.numpy as jnp
from jax import lax
from jax.experimental import pallas as pl
from jax.experimental.pallas import tpu as pltpu
from jax.experimental.pallas import tpu_sc as plsc

N = 10000
E = 320000
DIN = 128
DH = 128
DOUT = 64

NC = 2
NS = 16
EPT = E // (NC * NS)
RPT = 624
REM = N - NS * RPT
DEGW = 8

_mesh = plsc.VectorSubcoreMesh(core_axis_name="c", subcore_axis_name="s")


def _make_sc_agg(d, with_deg, eb, nbuf, ch):
    nb = EPT // eb
    nch = nb // ch
    out_type = [jax.ShapeDtypeStruct((NC * N, d), jnp.float32)]
    scratch = [
        pltpu.VMEM_SHARED((N, d), jnp.float32),
        pltpu.VMEM((2, ch, eb), jnp.int32),
        pltpu.VMEM((nb, eb), jnp.int32),
        pltpu.VMEM((nbuf, eb, d), jnp.float32),
        pltpu.SemaphoreType.DMA,
        pltpu.SemaphoreType.DMA,
    ]
    scratch += [pltpu.SemaphoreType.DMA] * (2 * nbuf)
    if with_deg:
        out_type.append(jax.ShapeDtypeStruct((NC * N, DEGW), jnp.float32))
        scratch += [
            pltpu.VMEM_SHARED((N, DEGW), jnp.float32),
            pltpu.VMEM((eb, DEGW), jnp.float32),
            pltpu.SemaphoreType.DMA,
        ]

    def sc_agg(table_hbm, edge_hbm, zrow_hbm, zdeg_hbm, ones_hbm, *rest):
        if with_deg:
            (msg_hbm, deg_hbm, acc, src_v, dst_v, rows_v, cs_s, cs_d,
             *sems) = rest
            sems, (dacc, ones_v, osem) = sems[:2 * nbuf], sems[2 * nbuf:]
        else:
            (msg_hbm, acc, src_v, dst_v, rows_v, cs_s, cs_d, *sems) = rest
        gsem = sems[:nbuf]
        ssem = sems[nbuf:2 * nbuf]
        c = lax.axis_index("c")
        s = lax.axis_index("s")
        tid = c * NS + s

        row0 = s * RPT
        pltpu.sync_copy(zrow_hbm.at[pl.ds(0, RPT)], acc.at[pl.ds(row0, RPT)])
        if with_deg:
            pltpu.sync_copy(zdeg_hbm.at[pl.ds(0, RPT)],
                            dacc.at[pl.ds(row0, RPT)])
            pltpu.sync_copy(ones_hbm, ones_v)

        @pl.when(s == NS - 1)
        def _zero_rem():
            pltpu.sync_copy(zrow_hbm.at[pl.ds(0, REM)],
                            acc.at[pl.ds(NS * RPT, REM)])
            if with_deg:
                pltpu.sync_copy(zdeg_hbm.at[pl.ds(0, REM)],
                                dacc.at[pl.ds(NS * RPT, REM)])

        plsc.subcore_barrier()

        pend_dst = pltpu.async_copy(edge_hbm.at[1, tid], dst_v, cs_d)
        pltpu.sync_copy(edge_hbm.at[0, tid, pl.ds(0, ch)], src_v.at[0])

        pend_scatter = [None] * nbuf
        pend_gather = [None] * nbuf
        pend_chunk = None
        chunk_ready = 0
        pend_ones = None

        def fire_gather(gn):
            nonlocal pend_chunk, chunk_ready
            kn = gn // ch
            if kn > chunk_ready:
                pend_chunk.wait()
                pend_chunk = None
                chunk_ready = kn
            bn = gn % nbuf
            if pend_scatter[bn] is not None:
                pend_scatter[bn].wait()
                pend_scatter[bn] = None
            pend_gather[bn] = pltpu.async_copy(
                table_hbm.at[src_v.at[kn % 2].at[gn % ch]], rows_v.at[bn],
                gsem[bn])

        for g in range(nb):
            k, j = divmod(g, ch)
            b = g % nbuf
            if j == 0 and k + 1 < nch:
                pend_chunk = pltpu.async_copy(
                    edge_hbm.at[0, tid, pl.ds((k + 1) * ch, ch)],
                    src_v.at[(k + 1) % 2], cs_s)
            if pend_gather[b] is None:
                fire_gather(g)
            if g == 0:
                pend_dst.wait()
            pend_gather[b].wait()
            pend_gather[b] = None
            for a in range(1, nbuf):
                gn = g + a
                if (gn < nb and gn // ch <= k + 1
                        and pend_gather[gn % nbuf] is None):
                    fire_gather(gn)
            pend_scatter[b] = pltpu.async_copy(
                rows_v.at[b], acc.at[dst_v.at[g]], ssem[b], add=True)
            if with_deg:
                if pend_ones is not None:
                    pend_ones.wait()
                pend_ones = pltpu.async_copy(
                    ones_v, dacc.at[dst_v.at[g]], osem, add=True)
        for p in pend_scatter:
            if p is not None:
                p.wait()
        if pend_ones is not None:
            pend_ones.wait()
        plsc.subcore_barrier()

        out0 = c * N + row0
        pltpu.sync_copy(acc.at[pl.ds(row0, RPT)], msg_hbm.at[pl.ds(out0, RPT)])
        if with_deg:
            pltpu.sync_copy(dacc.at[pl.ds(row0, RPT)],
                            deg_hbm.at[pl.ds(out0, RPT)])

        @pl.when(s == NS - 1)
        def _drain_rem():
            pltpu.sync_copy(acc.at[pl.ds(NS * RPT, REM)],
                            msg_hbm.at[pl.ds(c * N + NS * RPT, REM)])
            if with_deg:
                pltpu.sync_copy(dacc.at[pl.ds(NS * RPT, REM)],
                                deg_hbm.at[pl.ds(c * N + NS * RPT, REM)])

    return pl.kernel(
        sc_agg, mesh=_mesh, out_type=out_type, scratch_types=scratch,
        compiler_params=pltpu.CompilerParams(use_tc_tiling_on_sc=False))


EB, CH = 80, 5
NBUF1 = 3
NBUF2 = 6
_sc_agg128 = _make_sc_agg(DIN, True, EB, NBUF1, CH)
_sc_agg64 = _make_sc_agg(DOUT, False, EB, NBUF2, CH)

BN = 2000
BNP = BN // 2
PDB = BN * DEGW // 128
PDROWS = N * DEGW // 128
PYB = BN * DOUT // 128


def _deg_pair_cols(deg_ref, i):
    d = (deg_ref[pl.ds(i * PDB, PDB), :]
         + deg_ref[pl.ds(PDROWS + i * PDB, PDB), :])
    r_ = lax.broadcasted_iota(jnp.int32, (BNP, PDB), 0)
    c_ = lax.broadcasted_iota(jnp.int32, (BNP, PDB), 1)
    sel = jnp.where(c_ == r_ // 8, 1.0, 0.0)
    t = lax.dot_general(sel, d, (((1,), (0,)), ((), ())),
                        preferred_element_type=jnp.float32)
    rr = lax.broadcasted_iota(jnp.int32, (BNP, 128), 0)
    ll = lax.broadcasted_iota(jnp.int32, (BNP, 128), 1)
    de = jnp.sum(jnp.where(ll == (rr % 8) * 16, t, 0.0), axis=1,
                 keepdims=True)
    do = jnp.sum(jnp.where(ll == (rr % 8) * 16 + 8, t, 0.0), axis=1,
                 keepdims=True)
    return de, do


def _deg_col(deg_ref, i):
    d = (deg_ref[pl.ds(i * PDB, PDB), :]
         + deg_ref[pl.ds(PDROWS + i * PDB, PDB), :])
    r_ = lax.broadcasted_iota(jnp.int32, (BN, PDB), 0)
    c_ = lax.broadcasted_iota(jnp.int32, (BN, PDB), 1)
    sel = jnp.where(c_ == r_ // 16, 1.0, 0.0)
    t = lax.dot_general(sel, d, (((1,), (0,)), ((), ())),
                        preferred_element_type=jnp.float32)
    rr = lax.broadcasted_iota(jnp.int32, (BN, 128), 0)
    ll = lax.broadcasted_iota(jnp.int32, (BN, 128), 1)
    return jnp.sum(jnp.where(ll == (rr % 16) * 8, t, 0.0), axis=1,
                   keepdims=True)


def _tc_dense_body(msg_a, msg_b, deg_ref, x_ref, wl_ref, bl_ref,
                   wr_ref, wg_ref, y_ref):
    m = msg_a[...] + msg_b[...]
    deg = _deg_col(deg_ref, pl.program_id(0))
    mean = m / jnp.maximum(deg, 1.0)
    dn = (((1,), (1,)), ((), ()))
    h = lax.dot_general(mean, wl_ref[...], dn,
                        preferred_element_type=jnp.float32)
    h = h + bl_ref[...]
    h = h + lax.dot_general(x_ref[...], wr_ref[...], dn,
                            preferred_element_type=jnp.float32)
    h = jnp.maximum(h, 0.0)
    xt = lax.dot_general(h, wg_ref[...], dn,
                         preferred_element_type=jnp.float32)
    y_ref[...] = lax.rsqrt(deg + 1.0) * xt


_tc_dense = pl.pallas_call(
    _tc_dense_body,
    grid=(N // BN,),
    in_specs=[
        pl.BlockSpec((BN, DIN), lambda i: (i, 0)),
        pl.BlockSpec((BN, DIN), lambda i: (N // BN + i, 0)),
        pl.BlockSpec((NC * PDROWS, 128), lambda i: (0, 0)),
        pl.BlockSpec((BN, DIN), lambda i: (i, 0)),
        pl.BlockSpec((DH, DIN), lambda i: (0, 0)),
        pl.BlockSpec((1, DH), lambda i: (0, 0)),
        pl.BlockSpec((DH, DIN), lambda i: (0, 0)),
        pl.BlockSpec((DOUT, DH), lambda i: (0, 0)),
    ],
    out_specs=pl.BlockSpec((BN, DOUT), lambda i: (i, 0)),
    out_shape=jax.ShapeDtypeStruct((N, DOUT), jnp.float32),
)


def _tc_final_body(agg_a, agg_b, deg_ref, y_ref, bg_ref,
                   out_ref, soft_ref):
    i = pl.program_id(0)
    a2 = agg_a[...] + agg_b[...] + y_ref[...]
    de, do = _deg_pair_cols(deg_ref, i)
    outs, softs = [], []
    for half, dg in ((0, de), (1, do)):
        a = a2[:, half * DOUT:(half + 1) * DOUT]
        out = lax.rsqrt(dg + 1.0) * a + bg_ref[...]
        m = jnp.max(out, axis=1, keepdims=True)
        e = jnp.exp(out - m)
        outs.append(out)
        softs.append(e / jnp.sum(e, axis=1, keepdims=True))
    out_ref[...] = jnp.concatenate(outs, axis=1)
    soft_ref[...] = jnp.concatenate(softs, axis=1)


_tc_final = pl.pallas_call(
    _tc_final_body,
    grid=(N // BN,),
    in_specs=[
        pl.BlockSpec((PYB, 128), lambda i: (i, 0)),
        pl.BlockSpec((PYB, 128), lambda i: (N // BN + i, 0)),
        pl.BlockSpec((NC * PDROWS, 128), lambda i: (0, 0)),
        pl.BlockSpec((PYB, 128), lambda i: (i, 0)),
        pl.BlockSpec((1, DOUT), lambda i: (0, 0)),
    ],
    out_specs=[
        pl.BlockSpec((PYB, 128), lambda i: (i, 0)),
        pl.BlockSpec((PYB, 128), lambda i: (i, 0)),
    ],
    out_shape=[
        jax.ShapeDtypeStruct((N * DOUT // 128, 128), jnp.float32),
        jax.ShapeDtypeStruct((N * DOUT // 128, 128), jnp.float32),
    ],
)


def kernel(x, edge_index, W_sage_l, b_sage_l, W_sage_r, W_gcn, b_gcn):
    e4 = edge_index.reshape(2, NC * NS, EPT // EB, EB)
    zrow = jnp.zeros((RPT, DIN), jnp.float32)
    zdeg = jnp.zeros((RPT, DEGW), jnp.float32)
    ones = jnp.ones((EB, DEGW), jnp.float32)
    msg2, deg2 = _sc_agg128(x, e4, zrow, zdeg, ones)
    degp = deg2.reshape(NC * PDROWS, 128)
    y = _tc_dense(msg2, msg2, degp, x, W_sage_l,
                  b_sage_l.reshape(1, DH), W_sage_r, W_gcn)
    zrow64 = jnp.zeros((RPT, DOUT), jnp.float32)
    agg2 = _sc_agg64(y, e4, zrow64, zdeg, ones)[0]
    aggp = agg2.reshape(NC * N * DOUT // 128, 128)
    outp, softp = _tc_final(aggp, aggp, degp, y.reshape(N // 2, 128),
                            b_gcn.reshape(1, DOUT))
    return outp.reshape(N, DOUT), softp.reshape(N, DOUT)
